# baseline (speedup 1.0000x reference)
import functools

import jax
import jax.numpy as jnp
from jax import lax
from jax.experimental import pallas as pl
from jax.experimental.pallas import tpu as pltpu
from jax.experimental.pallas import tpu_sc as plsc

N_NODES = 10000
N_PAD = 10240
IN_CH = 128
OUT_SIZE = 32
N_EDGES = 320000
E_PAD = 330240
N_TILES = 32
N_SUB = 16
EPT = E_PAD // N_TILES
ROWS = EPT // 16
SUBCH = 5
SROWS = ROWS // SUBCH
DEG_ROWS = N_PAD // 16
DEG_CHUNKS = DEG_ROWS // 128
DEG_TSLICE = DEG_ROWS // N_SUB
HALF = N_PAD // 2
CORE_SLICE = HALF // N_SUB
ECH = 64
CAP = 2 * EPT + 96
CROWS64 = CAP // ECH
CH_HALF = IN_CH // 2


def _rsqrt_sc(v):
  i = lax.bitcast_convert_type(v, jnp.int32)
  i = jnp.int32(0x5F3759DF) - lax.shift_right_logical(i, 1)
  y = lax.bitcast_convert_type(i, jnp.float32)
  half = v * 0.5
  for _ in range(3):
    y = y * (1.5 - half * y * y)
  return y


def _bcast_lane(vec, j):
  dn = lax.GatherDimensionNumbers(
      offset_dims=(), collapsed_slice_dims=(0,), start_index_map=(0,))
  idx = jnp.full((16, 1), j, dtype=jnp.int32)
  return lax.gather(vec, idx, dn, (1,),
                    mode=lax.GatherScatterMode.PROMISE_IN_BOUNDS)


def _sc_body(src_hbm, dst_hbm, ew_hbm, x0_hbm, x1_hbm, out_hbm,
             src_v, dst_v, ewn_v, packc_v, ewnc_v, dloc_v,
             sidxa_v, sidxb_v,
             deg_v, rowsa_v, rowsb_v, scata_v, scatb_v, iidx_v,
             sema, semb, ssema, ssemb,
             agg_sh, degs_sh):
  c = lax.axis_index("c")
  s = lax.axis_index("s")
  lo = (c * HALF).astype(jnp.int32)

  zeros16 = jnp.zeros((16,), jnp.float32)
  izeros16 = jnp.zeros((16,), jnp.int32)
  lane = jnp.arange(16, dtype=jnp.int32)

  @pl.loop(0, ECH)
  def _(j):
    for r in range(CH_HALF // 16):
      rowsa_v[j, pl.ds(r * 16, 16)] = zeros16

  @pl.loop(0, CORE_SLICE // ECH)
  def _(b):
    pltpu.sync_copy(rowsa_v,
                    agg_sh.at[pl.ds(s * CORE_SLICE + b * ECH, ECH)])

  @pl.loop(0, DEG_ROWS)
  def _(i):
    deg_v[i, :] = zeros16

  @pl.loop(0, DEG_CHUNKS)
  def _(ch):
    for g in range(8):
      iidx_v[ch, pl.ds(g * 16, 16)] = lane + ch * 128 + g * 16

  pltpu.sync_copy(deg_v.at[pl.ds(s * DEG_TSLICE, DEG_TSLICE)],
                  degs_sh.at[pl.ds(s * DEG_TSLICE, DEG_TSLICE)])
  plsc.subcore_barrier()

  off = jnp.int32(0)
  for h in range(2 * SUBCH):
    pltpu.sync_copy(src_hbm.at[2 * SUBCH * s + h], src_v)
    pltpu.sync_copy(dst_hbm.at[2 * SUBCH * s + h], dst_v)
    pltpu.sync_copy(ew_hbm.at[2 * SUBCH * s + h], ewn_v)

    def _compact(i, off):
      sv = src_v[i, :]
      dv = dst_v[i, :]
      wv = ewn_v[i, :]
      plsc.addupdate_scatter(
          deg_v, [lax.shift_right_logical(dv, 4), dv & 15], wv)
      mask = (dv >= lo) & (dv < lo + HALF)
      plsc.store_compressed(packc_v.at[pl.ds(off, 16)],
                            sv | lax.shift_left(dv, 14), mask=mask)
      plsc.store_compressed(ewnc_v.at[pl.ds(off, 16)], wv, mask=mask)
      return off + jnp.sum(mask.astype(jnp.int32))

    off = pl.loop(0, SROWS, init_carry=off)(_compact)

  for k in range(ECH // 16):
    packc_v[pl.ds(off + k * 16, 16)] = izeros16 + lax.shift_left(lo, 14)
    ewnc_v[pl.ds(off + k * 16, 16)] = zeros16
  nch64 = lax.div(off + ECH - 1, jnp.int32(ECH))

  for ch in range(DEG_CHUNKS):
    pltpu.sync_copy(deg_v.at[pl.ds(ch * 128, 128)],
                    degs_sh.at[iidx_v.at[ch]], add=True)
  plsc.subcore_barrier()

  pltpu.sync_copy(degs_sh, deg_v)

  @pl.loop(0, DEG_ROWS)
  def _(i):
    deg_v[i, :] = _rsqrt_sc(deg_v[i, :])

  @pl.loop(0, nch64 * (ECH // 16))
  def _(i):
    pv = packc_v[pl.ds(i * 16, 16)]
    sv = pv & 16383
    dv = lax.shift_right_logical(pv, 14)
    w = ewnc_v[pl.ds(i * 16, 16)]
    dsrc = plsc.load_gather(
        deg_v, [lax.shift_right_logical(sv, 4), sv & 15])
    ddst = plsc.load_gather(
        deg_v, [lax.shift_right_logical(dv, 4), dv & 15])
    ewnc_v[pl.ds(i * 16, 16)] = dsrc * w * ddst

  def _g_start(xp_hbm, chunk, buf, sem, sidx):
    for k in range(ECH // 16):
      pv = packc_v[pl.ds(chunk * ECH + k * 16, 16)]
      sidx[pl.ds(k * 16, 16)] = pv & 16383
    pltpu.async_copy(xp_hbm.at[sidx], buf, sem)

  def _g_wait(xp_hbm, buf, sem, sidx):
    pltpu.make_async_copy(xp_hbm.at[sidx], buf, sem).wait()

  def _scale(chunk, buf, sbuf):
    for k in range(ECH // 16):
      nv = ewnc_v[pl.ds(chunk * ECH + k * 16, 16)]
      for j in range(16):
        b = _bcast_lane(nv, j)
        row = k * 16 + j
        for r in range(CH_HALF // 16):
          sbuf[row, pl.ds(r * 16, 16)] = buf[row, pl.ds(r * 16, 16)] * b

  def _s_start(chunk, sbuf, sem, dlrow):
    for k in range(ECH // 16):
      pv = packc_v[pl.ds(chunk * ECH + k * 16, 16)]
      dloc_v[dlrow, pl.ds(k * 16, 16)] = lax.shift_right_logical(pv, 14) - lo
    pltpu.async_copy(sbuf, agg_sh.at[dloc_v.at[dlrow]], sem, add=True)

  def _s_wait(sbuf, sem):
    pltpu.make_async_copy(sbuf, agg_sh.at[dloc_v.at[0]], sem).wait()

  for p, xp_hbm in enumerate((x0_hbm, x1_hbm)):

    @pl.when(nch64 > 0)
    def _():
      _g_start(xp_hbm, 0, rowsa_v, sema, sidxa_v)

    @pl.loop(0, lax.div(nch64 + 1, jnp.int32(2)))
    def _(i):
      a = 2 * i
      _g_wait(xp_hbm, rowsa_v, sema, sidxa_v)

      @pl.when(a + 1 < nch64)
      def _():
        _g_start(xp_hbm, a + 1, rowsb_v, semb, sidxb_v)

      @pl.when(i > 0)
      def _():
        _s_wait(scata_v, ssema)

      _scale(a, rowsa_v, scata_v)
      _s_start(a, scata_v, ssema, 0)

      @pl.when(a + 1 < nch64)
      def _():
        _g_wait(xp_hbm, rowsb_v, semb, sidxb_v)

        @pl.when(a + 2 < nch64)
        def _():
          _g_start(xp_hbm, a + 2, rowsa_v, sema, sidxa_v)

        @pl.when(i > 0)
        def _():
          _s_wait(scatb_v, ssemb)

        _scale(a + 1, rowsb_v, scatb_v)
        _s_start(a + 1, scatb_v, ssemb, 1)

    @pl.when(nch64 > 0)
    def _():
      _s_wait(scata_v, ssema)

    @pl.when(nch64 > 1)
    def _():
      _s_wait(scatb_v, ssemb)

    plsc.subcore_barrier()

    pltpu.sync_copy(agg_sh.at[pl.ds(s * CORE_SLICE, CORE_SLICE)],
                    out_hbm.at[p, pl.ds(c * HALF + s * CORE_SLICE,
                                        CORE_SLICE)])

    if p == 0:
      @pl.loop(0, ECH)
      def _(j):
        for r in range(CH_HALF // 16):
          rowsa_v[j, pl.ds(r * 16, 16)] = jnp.zeros((16,), jnp.float32)

      @pl.loop(0, CORE_SLICE // ECH)
      def _(b):
        pltpu.sync_copy(rowsa_v,
                        agg_sh.at[pl.ds(s * CORE_SLICE + b * ECH, ECH)])

      plsc.subcore_barrier()


def _sc_aggregate(src3, dst3, ew3, x0, x1):
  mesh = plsc.VectorSubcoreMesh(core_axis_name="c", subcore_axis_name="s")
  return pl.kernel(
      _sc_body,
      out_type=jax.ShapeDtypeStruct((2, N_PAD, CH_HALF), jnp.float32),
      mesh=mesh,
      scratch_types=[
          pltpu.VMEM((SROWS, 16), jnp.int32),
          pltpu.VMEM((SROWS, 16), jnp.int32),
          pltpu.VMEM((SROWS, 16), jnp.float32),
          pltpu.VMEM((CAP,), jnp.int32),
          pltpu.VMEM((CAP,), jnp.float32),
          pltpu.VMEM((2, ECH), jnp.int32),
          pltpu.VMEM((ECH,), jnp.int32),
          pltpu.VMEM((ECH,), jnp.int32),
          pltpu.VMEM((DEG_ROWS, 16), jnp.float32),
          pltpu.VMEM((ECH, CH_HALF), jnp.float32),
          pltpu.VMEM((ECH, CH_HALF), jnp.float32),
          pltpu.VMEM((ECH, CH_HALF), jnp.float32),
          pltpu.VMEM((ECH, CH_HALF), jnp.float32),
          pltpu.VMEM((DEG_CHUNKS, 128), jnp.int32),
          pltpu.SemaphoreType.DMA,
          pltpu.SemaphoreType.DMA,
          pltpu.SemaphoreType.DMA,
          pltpu.SemaphoreType.DMA,
          pltpu.VMEM_SHARED((HALF, CH_HALF), jnp.float32),
          pltpu.VMEM_SHARED((DEG_ROWS, 16), jnp.float32),
      ],
      compiler_params=pltpu.CompilerParams(
          needs_layout_passes=False, use_tc_tiling_on_sc=False),
      name="tgcn_sc_aggregate",
  )(src3, dst3, ew3, x0, x1)


def _dense_body(agg0_ref, agg1_ref, wz_ref, wh_ref, wlz_ref, wlh_ref,
                bz_ref, blz_ref, bh_ref, blh_ref, wo_ref, bo_ref, out_ref):
  agg = jnp.concatenate([agg0_ref[...], agg1_ref[...]], axis=1)
  mz = jnp.dot(wz_ref[...], wlz_ref[...], preferred_element_type=jnp.float32)
  mh = jnp.dot(wh_ref[...], wlh_ref[...], preferred_element_type=jnp.float32)
  cz = jnp.dot(bz_ref[...], wlz_ref[...],
               preferred_element_type=jnp.float32) + blz_ref[...]
  ch = jnp.dot(bh_ref[...], wlh_ref[...],
               preferred_element_type=jnp.float32) + blh_ref[...]
  z = jax.nn.sigmoid(
      jnp.dot(agg, mz, preferred_element_type=jnp.float32) + cz)
  ht = jnp.tanh(
      jnp.dot(agg, mh, preferred_element_type=jnp.float32) + ch)
  hn = jax.nn.relu((1.0 - z) * ht)
  out_ref[...] = (
      jnp.dot(hn, wo_ref[...], preferred_element_type=jnp.float32)
      + bo_ref[...])


def _dense(agg0, agg1, W_z, W_h, Wlz1, Wlh1, bz, blz, bh, blh, W_out, b_out):
  blk = 2000
  grid = (N_NODES // blk,)
  half_spec = pl.BlockSpec((blk, CH_HALF), lambda i: (i, 0))
  full = lambda shape: pl.BlockSpec(shape, lambda i: (0,) * len(shape))
  return pl.pallas_call(
      _dense_body,
      grid=grid,
      in_specs=[
          half_spec, half_spec,
          full((IN_CH, IN_CH)), full((IN_CH, IN_CH)),
          full((IN_CH, IN_CH)), full((IN_CH, IN_CH)),
          full((1, IN_CH)), full((1, IN_CH)),
          full((1, IN_CH)), full((1, IN_CH)),
          full((IN_CH, OUT_SIZE)), full((1, OUT_SIZE)),
      ],
      out_specs=pl.BlockSpec((blk, OUT_SIZE), lambda i: (i, 0)),
      out_shape=jax.ShapeDtypeStruct((N_NODES, OUT_SIZE), jnp.float32),
  )(agg0, agg1, W_z, W_h, Wlz1, Wlh1, bz, blz, bh, blh, W_out, b_out)


@jax.jit
def kernel(x, edge_index, edge_weight, W_z, b_z, W_r, b_r, W_h, b_h,
           Wl_z, bl_z, Wl_r, bl_r, Wl_h, bl_h, W_out, b_out):
  del W_r, b_r, Wl_r, bl_r

  pad = E_PAD - N_EDGES - N_NODES
  loops = jnp.arange(N_NODES, dtype=jnp.int32)
  zpad_i = jnp.zeros((pad,), jnp.int32)
  src = jnp.concatenate([edge_index[0].astype(jnp.int32), loops, zpad_i])
  dst = jnp.concatenate([edge_index[1].astype(jnp.int32), loops, zpad_i])
  ew = jnp.concatenate([edge_weight.astype(jnp.float32),
                        jnp.ones((N_NODES,), jnp.float32),
                        jnp.zeros((pad,), jnp.float32)])
  src3 = src.reshape(N_TILES * SUBCH, SROWS, 16)
  dst3 = dst.reshape(N_TILES * SUBCH, SROWS, 16)
  ew3 = ew.reshape(N_TILES * SUBCH, SROWS, 16)
  x_pad = jnp.zeros((N_PAD, IN_CH), jnp.float32).at[:N_NODES].set(x)

  agg = _sc_aggregate(src3, dst3, ew3,
                      x_pad[:, :CH_HALF], x_pad[:, CH_HALF:])

  Wlz1 = Wl_z[:IN_CH]
  Wlh1 = Wl_h[:IN_CH]

  return _dense(agg[0, :N_NODES], agg[1, :N_NODES], W_z, W_h, Wlz1, Wlh1,
                b_z.reshape(1, IN_CH), bl_z.reshape(1, IN_CH),
                b_h.reshape(1, IN_CH), bl_h.reshape(1, IN_CH),
                W_out, b_out.reshape(1, OUT_SIZE))


# speedup vs baseline: 1.0009x; 1.0009x over previous
import functools

import jax
import jax.numpy as jnp
from jax import lax
from jax.experimental import pallas as pl
from jax.experimental.pallas import tpu as pltpu
from jax.experimental.pallas import tpu_sc as plsc

N_NODES = 10000
N_PAD = 10240
IN_CH = 128
OUT_SIZE = 32
N_EDGES = 320000
E_PAD = 330240
N_TILES = 32
N_SUB = 16
EPT = E_PAD // N_TILES
ROWS = EPT // 16
SUBCH = 5
SROWS = ROWS // SUBCH
DEG_ROWS = N_PAD // 16
DEG_CHUNKS = DEG_ROWS // 128
DEG_TSLICE = DEG_ROWS // N_SUB
HALF = N_PAD // 2
CORE_SLICE = HALF // N_SUB
ECH = 64
CAP = 2 * EPT + 96
CROWS64 = CAP // ECH
CH_HALF = IN_CH // 2


def _rsqrt_sc(v):
  i = lax.bitcast_convert_type(v, jnp.int32)
  i = jnp.int32(0x5F3759DF) - lax.shift_right_logical(i, 1)
  y = lax.bitcast_convert_type(i, jnp.float32)
  half = v * 0.5
  for _ in range(3):
    y = y * (1.5 - half * y * y)
  return y


def _bcast_lane(vec, j):
  dn = lax.GatherDimensionNumbers(
      offset_dims=(), collapsed_slice_dims=(0,), start_index_map=(0,))
  idx = jnp.full((16, 1), j, dtype=jnp.int32)
  return lax.gather(vec, idx, dn, (1,),
                    mode=lax.GatherScatterMode.PROMISE_IN_BOUNDS)


def _sc_body(src_hbm, dst_hbm, ew_hbm, x0_hbm, x1_hbm, out_hbm,
             src_v, dst_v, ewn_v, packc_v, ewnc_v, dloc_v,
             sidxa_v, sidxb_v,
             deg_v, rowsa_v, rowsb_v, scata_v, scatb_v, iidx_v,
             sema, semb, ssema, ssemb,
             agg_sh, degs_sh):
  c = lax.axis_index("c")
  s = lax.axis_index("s")
  lo = (c * HALF).astype(jnp.int32)

  zeros16 = jnp.zeros((16,), jnp.float32)
  izeros16 = jnp.zeros((16,), jnp.int32)
  lane = jnp.arange(16, dtype=jnp.int32)

  @pl.loop(0, ECH)
  def _(j):
    for r in range(CH_HALF // 16):
      rowsa_v[j, pl.ds(r * 16, 16)] = zeros16

  @pl.loop(0, CORE_SLICE // ECH)
  def _(b):
    pltpu.sync_copy(rowsa_v,
                    agg_sh.at[pl.ds(s * CORE_SLICE + b * ECH, ECH)])

  @pl.loop(0, DEG_ROWS)
  def _(i):
    deg_v[i, :] = zeros16

  @pl.loop(0, DEG_CHUNKS)
  def _(ch):
    for g in range(8):
      iidx_v[ch, pl.ds(g * 16, 16)] = lane + ch * 128 + g * 16

  pltpu.sync_copy(deg_v.at[pl.ds(s * DEG_TSLICE, DEG_TSLICE)],
                  degs_sh.at[pl.ds(s * DEG_TSLICE, DEG_TSLICE)])
  plsc.subcore_barrier()

  off = jnp.int32(0)
  with jax.named_scope("ph_compact"):
    for h in range(2 * SUBCH):
      pltpu.sync_copy(src_hbm.at[2 * SUBCH * s + h], src_v)
      pltpu.sync_copy(dst_hbm.at[2 * SUBCH * s + h], dst_v)
      pltpu.sync_copy(ew_hbm.at[2 * SUBCH * s + h], ewn_v)

      def _compact(i, off):
        sv = src_v[i, :]
        dv = dst_v[i, :]
        wv = ewn_v[i, :]
        plsc.addupdate_scatter(
            deg_v, [lax.shift_right_logical(dv, 4), dv & 15], wv)
        mask = (dv >= lo) & (dv < lo + HALF)
        plsc.store_compressed(packc_v.at[pl.ds(off, 16)],
                              sv | lax.shift_left(dv, 14), mask=mask)
        plsc.store_compressed(ewnc_v.at[pl.ds(off, 16)], wv, mask=mask)
        return off + jnp.sum(mask.astype(jnp.int32))

      off = pl.loop(0, SROWS, init_carry=off)(_compact)

    for k in range(ECH // 16):
      packc_v[pl.ds(off + k * 16, 16)] = izeros16 + lax.shift_left(lo, 14)
      ewnc_v[pl.ds(off + k * 16, 16)] = zeros16
  nch64 = lax.div(off + ECH - 1, jnp.int32(ECH))

  with jax.named_scope("ph_degmerge"):
    for ch in range(DEG_CHUNKS):
      pltpu.sync_copy(deg_v.at[pl.ds(ch * 128, 128)],
                      degs_sh.at[iidx_v.at[ch]], add=True)
    plsc.subcore_barrier()

    pltpu.sync_copy(degs_sh, deg_v)

    @pl.loop(0, DEG_ROWS)
    def _(i):
      deg_v[i, :] = _rsqrt_sc(deg_v[i, :])

  with jax.named_scope("ph_norms"):
    @pl.loop(0, nch64 * (ECH // 16))
    def _(i):
      pv = packc_v[pl.ds(i * 16, 16)]
      sv = pv & 16383
      dv = lax.shift_right_logical(pv, 14)
      w = ewnc_v[pl.ds(i * 16, 16)]
      dsrc = plsc.load_gather(
          deg_v, [lax.shift_right_logical(sv, 4), sv & 15])
      ddst = plsc.load_gather(
          deg_v, [lax.shift_right_logical(dv, 4), dv & 15])
      ewnc_v[pl.ds(i * 16, 16)] = dsrc * w * ddst

  def _g_start(xp_hbm, chunk, buf, sem, sidx):
    for k in range(ECH // 16):
      pv = packc_v[pl.ds(chunk * ECH + k * 16, 16)]
      sidx[pl.ds(k * 16, 16)] = pv & 16383
    pltpu.async_copy(xp_hbm.at[sidx], buf, sem)

  def _g_wait(xp_hbm, buf, sem, sidx):
    pltpu.make_async_copy(xp_hbm.at[sidx], buf, sem).wait()

  def _scale(chunk, buf, sbuf):
    for k in range(ECH // 16):
      nv = ewnc_v[pl.ds(chunk * ECH + k * 16, 16)]
      for j in range(16):
        b = _bcast_lane(nv, j)
        row = k * 16 + j
        for r in range(CH_HALF // 16):
          sbuf[row, pl.ds(r * 16, 16)] = buf[row, pl.ds(r * 16, 16)] * b

  def _s_start(chunk, sbuf, sem, dlrow):
    for k in range(ECH // 16):
      pv = packc_v[pl.ds(chunk * ECH + k * 16, 16)]
      dloc_v[dlrow, pl.ds(k * 16, 16)] = lax.shift_right_logical(pv, 14) - lo
    pltpu.async_copy(sbuf, agg_sh.at[dloc_v.at[dlrow]], sem, add=True)

  def _s_wait(sbuf, sem):
    pltpu.make_async_copy(sbuf, agg_sh.at[dloc_v.at[0]], sem).wait()

  for p, xp_hbm in enumerate((x0_hbm, x1_hbm)):
    scope = jax.named_scope(f"ph_main{p}")
    scope.__enter__()

    @pl.when(nch64 > 0)
    def _():
      _g_start(xp_hbm, 0, rowsa_v, sema, sidxa_v)

    @pl.loop(0, lax.div(nch64 + 1, jnp.int32(2)))
    def _(i):
      a = 2 * i
      _g_wait(xp_hbm, rowsa_v, sema, sidxa_v)

      @pl.when(a + 1 < nch64)
      def _():
        _g_start(xp_hbm, a + 1, rowsb_v, semb, sidxb_v)

      @pl.when(i > 0)
      def _():
        _s_wait(scata_v, ssema)

      _scale(a, rowsa_v, scata_v)
      _s_start(a, scata_v, ssema, 0)

      @pl.when(a + 1 < nch64)
      def _():
        _g_wait(xp_hbm, rowsb_v, semb, sidxb_v)

        @pl.when(a + 2 < nch64)
        def _():
          _g_start(xp_hbm, a + 2, rowsa_v, sema, sidxa_v)

        @pl.when(i > 0)
        def _():
          _s_wait(scatb_v, ssemb)

        _scale(a + 1, rowsb_v, scatb_v)
        _s_start(a + 1, scatb_v, ssemb, 1)

    @pl.when(nch64 > 0)
    def _():
      _s_wait(scata_v, ssema)

    @pl.when(nch64 > 1)
    def _():
      _s_wait(scatb_v, ssemb)

    plsc.subcore_barrier()
    scope.__exit__(None, None, None)

    pltpu.sync_copy(agg_sh.at[pl.ds(s * CORE_SLICE, CORE_SLICE)],
                    out_hbm.at[p, pl.ds(c * HALF + s * CORE_SLICE,
                                        CORE_SLICE)])

    if p == 0:
      @pl.loop(0, ECH)
      def _(j):
        for r in range(CH_HALF // 16):
          rowsa_v[j, pl.ds(r * 16, 16)] = jnp.zeros((16,), jnp.float32)

      @pl.loop(0, CORE_SLICE // ECH)
      def _(b):
        pltpu.sync_copy(rowsa_v,
                        agg_sh.at[pl.ds(s * CORE_SLICE + b * ECH, ECH)])

      plsc.subcore_barrier()


def _sc_aggregate(src3, dst3, ew3, x0, x1):
  mesh = plsc.VectorSubcoreMesh(core_axis_name="c", subcore_axis_name="s")
  return pl.kernel(
      _sc_body,
      out_type=jax.ShapeDtypeStruct((2, N_PAD, CH_HALF), jnp.float32),
      mesh=mesh,
      scratch_types=[
          pltpu.VMEM((SROWS, 16), jnp.int32),
          pltpu.VMEM((SROWS, 16), jnp.int32),
          pltpu.VMEM((SROWS, 16), jnp.float32),
          pltpu.VMEM((CAP,), jnp.int32),
          pltpu.VMEM((CAP,), jnp.float32),
          pltpu.VMEM((2, ECH), jnp.int32),
          pltpu.VMEM((ECH,), jnp.int32),
          pltpu.VMEM((ECH,), jnp.int32),
          pltpu.VMEM((DEG_ROWS, 16), jnp.float32),
          pltpu.VMEM((ECH, CH_HALF), jnp.float32),
          pltpu.VMEM((ECH, CH_HALF), jnp.float32),
          pltpu.VMEM((ECH, CH_HALF), jnp.float32),
          pltpu.VMEM((ECH, CH_HALF), jnp.float32),
          pltpu.VMEM((DEG_CHUNKS, 128), jnp.int32),
          pltpu.SemaphoreType.DMA,
          pltpu.SemaphoreType.DMA,
          pltpu.SemaphoreType.DMA,
          pltpu.SemaphoreType.DMA,
          pltpu.VMEM_SHARED((HALF, CH_HALF), jnp.float32),
          pltpu.VMEM_SHARED((DEG_ROWS, 16), jnp.float32),
      ],
      compiler_params=pltpu.CompilerParams(
          needs_layout_passes=False, use_tc_tiling_on_sc=False),
      name="tgcn_sc_aggregate",
  )(src3, dst3, ew3, x0, x1)


def _dense_body(agg0_ref, agg1_ref, wz_ref, wh_ref, wlz_ref, wlh_ref,
                bz_ref, blz_ref, bh_ref, blh_ref, wo_ref, bo_ref, out_ref):
  agg = jnp.concatenate([agg0_ref[...], agg1_ref[...]], axis=1)
  mz = jnp.dot(wz_ref[...], wlz_ref[...], preferred_element_type=jnp.float32)
  mh = jnp.dot(wh_ref[...], wlh_ref[...], preferred_element_type=jnp.float32)
  cz = jnp.dot(bz_ref[...], wlz_ref[...],
               preferred_element_type=jnp.float32) + blz_ref[...]
  ch = jnp.dot(bh_ref[...], wlh_ref[...],
               preferred_element_type=jnp.float32) + blh_ref[...]
  z = jax.nn.sigmoid(
      jnp.dot(agg, mz, preferred_element_type=jnp.float32) + cz)
  ht = jnp.tanh(
      jnp.dot(agg, mh, preferred_element_type=jnp.float32) + ch)
  hn = jax.nn.relu((1.0 - z) * ht)
  out_ref[...] = (
      jnp.dot(hn, wo_ref[...], preferred_element_type=jnp.float32)
      + bo_ref[...])


def _dense(agg0, agg1, W_z, W_h, Wlz1, Wlh1, bz, blz, bh, blh, W_out, b_out):
  blk = 2000
  grid = (N_NODES // blk,)
  half_spec = pl.BlockSpec((blk, CH_HALF), lambda i: (i, 0))
  full = lambda shape: pl.BlockSpec(shape, lambda i: (0,) * len(shape))
  return pl.pallas_call(
      _dense_body,
      grid=grid,
      in_specs=[
          half_spec, half_spec,
          full((IN_CH, IN_CH)), full((IN_CH, IN_CH)),
          full((IN_CH, IN_CH)), full((IN_CH, IN_CH)),
          full((1, IN_CH)), full((1, IN_CH)),
          full((1, IN_CH)), full((1, IN_CH)),
          full((IN_CH, OUT_SIZE)), full((1, OUT_SIZE)),
      ],
      out_specs=pl.BlockSpec((blk, OUT_SIZE), lambda i: (i, 0)),
      out_shape=jax.ShapeDtypeStruct((N_NODES, OUT_SIZE), jnp.float32),
  )(agg0, agg1, W_z, W_h, Wlz1, Wlh1, bz, blz, bh, blh, W_out, b_out)


@jax.jit
def kernel(x, edge_index, edge_weight, W_z, b_z, W_r, b_r, W_h, b_h,
           Wl_z, bl_z, Wl_r, bl_r, Wl_h, bl_h, W_out, b_out):
  del W_r, b_r, Wl_r, bl_r

  pad = E_PAD - N_EDGES - N_NODES
  loops = jnp.arange(N_NODES, dtype=jnp.int32)
  zpad_i = jnp.zeros((pad,), jnp.int32)
  src = jnp.concatenate([edge_index[0].astype(jnp.int32), loops, zpad_i])
  dst = jnp.concatenate([edge_index[1].astype(jnp.int32), loops, zpad_i])
  ew = jnp.concatenate([edge_weight.astype(jnp.float32),
                        jnp.ones((N_NODES,), jnp.float32),
                        jnp.zeros((pad,), jnp.float32)])
  src3 = src.reshape(N_TILES * SUBCH, SROWS, 16)
  dst3 = dst.reshape(N_TILES * SUBCH, SROWS, 16)
  ew3 = ew.reshape(N_TILES * SUBCH, SROWS, 16)
  x_pad = jnp.zeros((N_PAD, IN_CH), jnp.float32).at[:N_NODES].set(x)

  agg = _sc_aggregate(src3, dst3, ew3,
                      x_pad[:, :CH_HALF], x_pad[:, CH_HALF:])

  Wlz1 = Wl_z[:IN_CH]
  Wlh1 = Wl_h[:IN_CH]

  return _dense(agg[0, :N_NODES], agg[1, :N_NODES], W_z, W_h, Wlz1, Wlh1,
                b_z.reshape(1, IN_CH), bl_z.reshape(1, IN_CH),
                b_h.reshape(1, IN_CH), bl_h.reshape(1, IN_CH),
                W_out, b_out.reshape(1, OUT_SIZE))


# speedup vs baseline: 1.3207x; 1.3196x over previous
import functools

import jax
import jax.numpy as jnp
import numpy as np
from jax import lax
from jax.experimental import pallas as pl
from jax.experimental.pallas import tpu as pltpu
from jax.experimental.pallas import tpu_sc as plsc

N_NODES = 10000
N_PAD = 10240
IN_CH = 128
OUT_SIZE = 32
N_EDGES = 320000
E_PAD = 330240
N_TILES = 32
N_SUB = 16
EPT = E_PAD // N_TILES
ROWS = EPT // 16
SUBCH = 5
SROWS = ROWS // SUBCH
DEG_ROWS = N_PAD // 16
DEG_CHUNKS = DEG_ROWS // 128
DEG_TSLICE = DEG_ROWS // N_SUB
HALF = N_PAD // 2
CORE_SLICE = HALF // N_SUB
ECH = 64
CAP = 2 * EPT + 96
CROWS64 = CAP // ECH
CH_HALF = IN_CH // 2


def _rsqrt_sc(v):
  i = lax.bitcast_convert_type(v, jnp.int32)
  i = jnp.int32(0x5F3759DF) - lax.shift_right_logical(i, 1)
  y = lax.bitcast_convert_type(i, jnp.float32)
  half = v * 0.5
  for _ in range(3):
    y = y * (1.5 - half * y * y)
  return y


def _bcast_lane(vec, j):
  dn = lax.GatherDimensionNumbers(
      offset_dims=(), collapsed_slice_dims=(0,), start_index_map=(0,))
  idx = jnp.full((16, 1), j, dtype=jnp.int32)
  return lax.gather(vec, idx, dn, (1,),
                    mode=lax.GatherScatterMode.PROMISE_IN_BOUNDS)


def _sc_body(src_hbm, dst_hbm, ew_hbm, x_hbm, out_hbm,
             src_v, dst_v, ewn_v, packc_v, ewnc_v, dloc_v,
             sidxa_v, sidxb_v,
             deg_v, rowsa_v, rowsb_v, scata_v, scatb_v, iidx_v,
             sema, semb, ssema, ssemb,
             agg_sh, degs_sh):
  c = lax.axis_index("c")
  s = lax.axis_index("s")
  lo = (c * HALF).astype(jnp.int32)

  zeros16 = jnp.zeros((16,), jnp.float32)
  izeros16 = jnp.zeros((16,), jnp.int32)
  lane = jnp.arange(16, dtype=jnp.int32)

  zeros32b = jnp.zeros((32,), jnp.bfloat16)

  @pl.loop(0, ECH)
  def _(j):
    for r in range(IN_CH // 32):
      scata_v[j, pl.ds(r * 32, 32)] = zeros32b

  @pl.loop(0, CORE_SLICE // ECH)
  def _(b):
    pltpu.sync_copy(scata_v,
                    agg_sh.at[pl.ds(s * CORE_SLICE + b * ECH, ECH)])

  @pl.loop(0, DEG_ROWS)
  def _(i):
    deg_v[i, :] = zeros16

  @pl.loop(0, DEG_CHUNKS)
  def _(ch):
    for g in range(8):
      iidx_v[ch, pl.ds(g * 16, 16)] = lane + ch * 128 + g * 16

  pltpu.sync_copy(deg_v.at[pl.ds(s * DEG_TSLICE, DEG_TSLICE)],
                  degs_sh.at[pl.ds(s * DEG_TSLICE, DEG_TSLICE)])
  plsc.subcore_barrier()

  off = jnp.int32(0)
  with jax.named_scope("ph_compact"):
    for h in range(2 * SUBCH):
      pltpu.sync_copy(src_hbm.at[2 * SUBCH * s + h], src_v)
      pltpu.sync_copy(dst_hbm.at[2 * SUBCH * s + h], dst_v)
      pltpu.sync_copy(ew_hbm.at[2 * SUBCH * s + h], ewn_v)

      def _compact(i, off):
        sv = src_v[i, :]
        dv = dst_v[i, :]
        wv = ewn_v[i, :]
        plsc.addupdate_scatter(
            deg_v, [lax.shift_right_logical(dv, 4), dv & 15], wv)
        mask = (dv >= lo) & (dv < lo + HALF)
        plsc.store_compressed(packc_v.at[pl.ds(off, 16)],
                              sv | lax.shift_left(dv, 14), mask=mask)
        plsc.store_compressed(ewnc_v.at[pl.ds(off, 16)], wv, mask=mask)
        return off + jnp.sum(mask.astype(jnp.int32))

      off = pl.loop(0, SROWS, init_carry=off)(_compact)

    for k in range(ECH // 16):
      packc_v[pl.ds(off + k * 16, 16)] = izeros16 + lax.shift_left(lo, 14)
      ewnc_v[pl.ds(off + k * 16, 16)] = zeros16
  nch64 = lax.div(off + ECH - 1, jnp.int32(ECH))

  with jax.named_scope("ph_degmerge"):
    for ch in range(DEG_CHUNKS):
      pltpu.sync_copy(deg_v.at[pl.ds(ch * 128, 128)],
                      degs_sh.at[iidx_v.at[ch]], add=True)
    plsc.subcore_barrier()

    pltpu.sync_copy(degs_sh, deg_v)

    @pl.loop(0, DEG_ROWS)
    def _(i):
      deg_v[i, :] = _rsqrt_sc(deg_v[i, :])

  with jax.named_scope("ph_norms"):
    @pl.loop(0, nch64 * (ECH // 16))
    def _(i):
      pv = packc_v[pl.ds(i * 16, 16)]
      sv = pv & 16383
      dv = lax.shift_right_logical(pv, 14)
      w = ewnc_v[pl.ds(i * 16, 16)]
      dsrc = plsc.load_gather(
          deg_v, [lax.shift_right_logical(sv, 4), sv & 15])
      ddst = plsc.load_gather(
          deg_v, [lax.shift_right_logical(dv, 4), dv & 15])
      ewnc_v[pl.ds(i * 16, 16)] = dsrc * w * ddst

  def _g_start(chunk, buf, sem, sidx):
    for k in range(ECH // 16):
      pv = packc_v[pl.ds(chunk * ECH + k * 16, 16)]
      sidx[pl.ds(k * 16, 16)] = pv & 16383
    pltpu.async_copy(x_hbm.at[sidx], buf, sem)

  def _g_wait(buf, sem, sidx):
    pltpu.make_async_copy(x_hbm.at[sidx], buf, sem).wait()

  def _scale(chunk, buf, sbuf):
    for k in range(ECH // 16):
      nv = ewnc_v[pl.ds(chunk * ECH + k * 16, 16)]
      for j in range(16):
        b = _bcast_lane(nv, j)
        row = k * 16 + j
        for r in range(IN_CH // 32):
          va = buf[row, pl.ds(r * 32, 16)] * b
          vb = buf[row, pl.ds(r * 32 + 16, 16)] * b
          sbuf[row, pl.ds(r * 32, 32)] = plsc.pack(
              va, vb, format=plsc.PackFormat.INTERLEAVED)

  def _s_start(chunk, sbuf, sem, dlrow):
    for k in range(ECH // 16):
      pv = packc_v[pl.ds(chunk * ECH + k * 16, 16)]
      dloc_v[dlrow, pl.ds(k * 16, 16)] = lax.shift_right_logical(pv, 14) - lo
    pltpu.async_copy(sbuf, agg_sh.at[dloc_v.at[dlrow]], sem, add=True)

  def _s_wait(sbuf, sem):
    pltpu.make_async_copy(sbuf, agg_sh.at[dloc_v.at[0]], sem).wait()

  @pl.when(nch64 > 0)
  def _():
    _g_start(0, rowsa_v, sema, sidxa_v)

  @pl.loop(0, lax.div(nch64 + 1, jnp.int32(2)))
  def _(i):
    a = 2 * i
    _g_wait(rowsa_v, sema, sidxa_v)

    @pl.when(a + 1 < nch64)
    def _():
      _g_start(a + 1, rowsb_v, semb, sidxb_v)

    @pl.when(i > 0)
    def _():
      _s_wait(scata_v, ssema)

    _scale(a, rowsa_v, scata_v)
    _s_start(a, scata_v, ssema, 0)

    @pl.when(a + 1 < nch64)
    def _():
      _g_wait(rowsb_v, semb, sidxb_v)

      @pl.when(a + 2 < nch64)
      def _():
        _g_start(a + 2, rowsa_v, sema, sidxa_v)

      @pl.when(i > 0)
      def _():
        _s_wait(scatb_v, ssemb)

      _scale(a + 1, rowsb_v, scatb_v)
      _s_start(a + 1, scatb_v, ssemb, 1)

  @pl.when(nch64 > 0)
  def _():
    _s_wait(scata_v, ssema)

  @pl.when(nch64 > 1)
  def _():
    _s_wait(scatb_v, ssemb)

  plsc.subcore_barrier()

  pltpu.sync_copy(agg_sh.at[pl.ds(s * CORE_SLICE, CORE_SLICE)],
                  out_hbm.at[pl.ds(c * HALF + s * CORE_SLICE, CORE_SLICE)])


def _sc_aggregate(src3, dst3, ew3, x_pad):
  mesh = plsc.VectorSubcoreMesh(core_axis_name="c", subcore_axis_name="s")
  return pl.kernel(
      _sc_body,
      out_type=jax.ShapeDtypeStruct((N_PAD, IN_CH), jnp.bfloat16),
      mesh=mesh,
      scratch_types=[
          pltpu.VMEM((SROWS, 16), jnp.int32),
          pltpu.VMEM((SROWS, 16), jnp.int32),
          pltpu.VMEM((SROWS, 16), jnp.float32),
          pltpu.VMEM((CAP,), jnp.int32),
          pltpu.VMEM((CAP,), jnp.float32),
          pltpu.VMEM((2, ECH), jnp.int32),
          pltpu.VMEM((ECH,), jnp.int32),
          pltpu.VMEM((ECH,), jnp.int32),
          pltpu.VMEM((DEG_ROWS, 16), jnp.float32),
          pltpu.VMEM((ECH, IN_CH), jnp.float32),
          pltpu.VMEM((ECH, IN_CH), jnp.float32),
          pltpu.VMEM((ECH, IN_CH), jnp.bfloat16),
          pltpu.VMEM((ECH, IN_CH), jnp.bfloat16),
          pltpu.VMEM((DEG_CHUNKS, 128), jnp.int32),
          pltpu.SemaphoreType.DMA,
          pltpu.SemaphoreType.DMA,
          pltpu.SemaphoreType.DMA,
          pltpu.SemaphoreType.DMA,
          pltpu.VMEM_SHARED((HALF, IN_CH), jnp.bfloat16),
          pltpu.VMEM_SHARED((DEG_ROWS, 16), jnp.float32),
      ],
      compiler_params=pltpu.CompilerParams(
          needs_layout_passes=False, use_tc_tiling_on_sc=False),
      name="tgcn_sc_aggregate",
  )(src3, dst3, ew3, x_pad)


def _dense_body(agg_ref, wz_ref, wh_ref, wlz_ref, wlh_ref,
                bz_ref, blz_ref, bh_ref, blh_ref, wo_ref, bo_ref, out_ref):
  agg = agg_ref[...].astype(jnp.float32)
  mz = jnp.dot(wz_ref[...], wlz_ref[...], preferred_element_type=jnp.float32)
  mh = jnp.dot(wh_ref[...], wlh_ref[...], preferred_element_type=jnp.float32)
  cz = jnp.dot(bz_ref[...], wlz_ref[...],
               preferred_element_type=jnp.float32) + blz_ref[...]
  ch = jnp.dot(bh_ref[...], wlh_ref[...],
               preferred_element_type=jnp.float32) + blh_ref[...]
  z = jax.nn.sigmoid(
      jnp.dot(agg, mz, preferred_element_type=jnp.float32) + cz)
  ht = jnp.tanh(
      jnp.dot(agg, mh, preferred_element_type=jnp.float32) + ch)
  hn = jax.nn.relu((1.0 - z) * ht)
  out_ref[...] = (
      jnp.dot(hn, wo_ref[...], preferred_element_type=jnp.float32)
      + bo_ref[...])


def _dense(agg, W_z, W_h, Wlz1, Wlh1, bz, blz, bh, blh, W_out, b_out):
  blk = 2000
  grid = (N_NODES // blk,)
  full = lambda shape: pl.BlockSpec(shape, lambda i: (0,) * len(shape))
  return pl.pallas_call(
      _dense_body,
      grid=grid,
      in_specs=[
          pl.BlockSpec((blk, IN_CH), lambda i: (i, 0)),
          full((IN_CH, IN_CH)), full((IN_CH, IN_CH)),
          full((IN_CH, IN_CH)), full((IN_CH, IN_CH)),
          full((1, IN_CH)), full((1, IN_CH)),
          full((1, IN_CH)), full((1, IN_CH)),
          full((IN_CH, OUT_SIZE)), full((1, OUT_SIZE)),
      ],
      out_specs=pl.BlockSpec((blk, OUT_SIZE), lambda i: (i, 0)),
      out_shape=jax.ShapeDtypeStruct((N_NODES, OUT_SIZE), jnp.float32),
  )(agg, W_z, W_h, Wlz1, Wlh1, bz, blz, bh, blh, W_out, b_out)


@jax.jit
def kernel(x, edge_index, edge_weight, W_z, b_z, W_r, b_r, W_h, b_h,
           Wl_z, bl_z, Wl_r, bl_r, Wl_h, bl_h, W_out, b_out):
  del W_r, b_r, Wl_r, bl_r

  pad = E_PAD - N_EDGES - N_NODES
  loops = jnp.arange(N_NODES, dtype=jnp.int32)
  zpad_i = jnp.zeros((pad,), jnp.int32)
  src = jnp.concatenate([edge_index[0].astype(jnp.int32), loops, zpad_i])
  dst = jnp.concatenate([edge_index[1].astype(jnp.int32), loops, zpad_i])
  ew = jnp.concatenate([edge_weight.astype(jnp.float32),
                        jnp.ones((N_NODES,), jnp.float32),
                        jnp.zeros((pad,), jnp.float32)])
  src3 = src.reshape(N_TILES * SUBCH, SROWS, 16)
  dst3 = dst.reshape(N_TILES * SUBCH, SROWS, 16)
  ew3 = ew.reshape(N_TILES * SUBCH, SROWS, 16)
  x_pad = jnp.zeros((N_PAD, IN_CH), jnp.float32).at[:N_NODES].set(x)

  agg = _sc_aggregate(src3, dst3, ew3, x_pad)

  Wlz1 = Wl_z[:IN_CH]
  Wlh1 = Wl_h[:IN_CH]

  perm = np.empty((IN_CH,), np.int32)
  for r in range(IN_CH // 32):
    for l in range(16):
      perm[32 * r + 2 * l] = 32 * r + l
      perm[32 * r + 2 * l + 1] = 32 * r + 16 + l

  return _dense(agg[:N_NODES], W_z[perm], W_h[perm], Wlz1, Wlh1,
                b_z.reshape(1, IN_CH), bl_z.reshape(1, IN_CH),
                b_h.reshape(1, IN_CH), bl_h.reshape(1, IN_CH),
                W_out, b_out.reshape(1, OUT_SIZE))


# speedup vs baseline: 1.4525x; 1.0998x over previous
import functools

import jax
import jax.numpy as jnp
import numpy as np
from jax import lax
from jax.experimental import pallas as pl
from jax.experimental.pallas import tpu as pltpu
from jax.experimental.pallas import tpu_sc as plsc

N_NODES = 10000
N_PAD = 10240
IN_CH = 128
OUT_SIZE = 32
N_EDGES = 320000
N_TILES = 32
N_SUB = 16
EPT = N_EDGES // N_TILES
ROWS = EPT // 16
SUBCH = 5
SROWS = ROWS // SUBCH
DEG_ROWS = N_PAD // 16
DEG_CHUNKS = DEG_ROWS // 128
DEG_TSLICE = DEG_ROWS // N_SUB
HALF = N_PAD // 2
CORE_SLICE = HALF // N_SUB
ECH = 64
CAP = 2 * EPT + 96
CROWS64 = CAP // ECH
CH_HALF = IN_CH // 2


def _rsqrt_sc(v):
  i = lax.bitcast_convert_type(v, jnp.int32)
  i = jnp.int32(0x5F3759DF) - lax.shift_right_logical(i, 1)
  y = lax.bitcast_convert_type(i, jnp.float32)
  half = v * 0.5
  for _ in range(3):
    y = y * (1.5 - half * y * y)
  return y


def _bcast_lane(vec, j):
  dn = lax.GatherDimensionNumbers(
      offset_dims=(), collapsed_slice_dims=(0,), start_index_map=(0,))
  idx = jnp.full((16, 1), j, dtype=jnp.int32)
  return lax.gather(vec, idx, dn, (1,),
                    mode=lax.GatherScatterMode.PROMISE_IN_BOUNDS)


def _sc_body(src_hbm, dst_hbm, ew_hbm, x_hbm, out_hbm, dis_hbm,
             src_v, dst_v, ewn_v, packc_v, ewnc_v, dloc_v,
             sidxa_v, sidxb_v,
             deg_v, rowsa_v, rowsb_v, scata_v, scatb_v, iidx_v,
             sema, semb, ssema, ssemb,
             agg_sh, degs_sh):
  c = lax.axis_index("c")
  s = lax.axis_index("s")
  lo = (c * HALF).astype(jnp.int32)

  zeros16 = jnp.zeros((16,), jnp.float32)
  izeros16 = jnp.zeros((16,), jnp.int32)
  lane = jnp.arange(16, dtype=jnp.int32)

  zeros32b = jnp.zeros((32,), jnp.bfloat16)

  @pl.loop(0, ECH)
  def _(j):
    for r in range(IN_CH // 32):
      scata_v[j, pl.ds(r * 32, 32)] = zeros32b

  @pl.loop(0, CORE_SLICE // ECH)
  def _(b):
    pltpu.sync_copy(scata_v,
                    agg_sh.at[pl.ds(s * CORE_SLICE + b * ECH, ECH)])

  @pl.loop(0, DEG_ROWS)
  def _(i):
    deg_v[i, :] = zeros16

  @pl.loop(0, DEG_CHUNKS)
  def _(ch):
    for g in range(8):
      iidx_v[ch, pl.ds(g * 16, 16)] = lane + ch * 128 + g * 16

  pltpu.sync_copy(deg_v.at[pl.ds(s * DEG_TSLICE, DEG_TSLICE)],
                  degs_sh.at[pl.ds(s * DEG_TSLICE, DEG_TSLICE)])
  plsc.subcore_barrier()

  off = jnp.int32(0)
  with jax.named_scope("ph_compact"):
    for h in range(2 * SUBCH):
      pltpu.sync_copy(src_hbm.at[2 * SUBCH * s + h], src_v)
      pltpu.sync_copy(dst_hbm.at[2 * SUBCH * s + h], dst_v)
      pltpu.sync_copy(ew_hbm.at[2 * SUBCH * s + h], ewn_v)

      def _compact(i, off):
        sv = src_v[i, :]
        dv = dst_v[i, :]
        wv = ewn_v[i, :]
        plsc.addupdate_scatter(
            deg_v, [lax.shift_right_logical(dv, 4), dv & 15], wv)
        mask = (dv >= lo) & (dv < lo + HALF)
        plsc.store_compressed(packc_v.at[pl.ds(off, 16)],
                              sv | lax.shift_left(dv, 14), mask=mask)
        plsc.store_compressed(ewnc_v.at[pl.ds(off, 16)], wv, mask=mask)
        return off + jnp.sum(mask.astype(jnp.int32))

      off = pl.loop(0, SROWS, init_carry=off)(_compact)

    for k in range(ECH // 16):
      packc_v[pl.ds(off + k * 16, 16)] = izeros16 + lax.shift_left(lo, 14)
      ewnc_v[pl.ds(off + k * 16, 16)] = zeros16
  nch64 = lax.div(off + ECH - 1, jnp.int32(ECH))

  with jax.named_scope("ph_degmerge"):
    for ch in range(DEG_CHUNKS):
      pltpu.sync_copy(deg_v.at[pl.ds(ch * 128, 128)],
                      degs_sh.at[iidx_v.at[ch]], add=True)
    plsc.subcore_barrier()

    pltpu.sync_copy(degs_sh, deg_v)

    @pl.loop(0, DEG_ROWS)
    def _(i):
      deg_v[i, :] = _rsqrt_sc(deg_v[i, :] + 1.0)

    @pl.when(c == 0)
    def _():
      pltpu.sync_copy(deg_v.at[pl.ds(s * DEG_TSLICE, DEG_TSLICE)],
                      dis_hbm.at[pl.ds(s * DEG_TSLICE, DEG_TSLICE)])

  with jax.named_scope("ph_norms"):
    @pl.loop(0, nch64 * (ECH // 16))
    def _(i):
      pv = packc_v[pl.ds(i * 16, 16)]
      sv = pv & 16383
      dv = lax.shift_right_logical(pv, 14)
      w = ewnc_v[pl.ds(i * 16, 16)]
      dsrc = plsc.load_gather(
          deg_v, [lax.shift_right_logical(sv, 4), sv & 15])
      ddst = plsc.load_gather(
          deg_v, [lax.shift_right_logical(dv, 4), dv & 15])
      ewnc_v[pl.ds(i * 16, 16)] = dsrc * w * ddst

  def _g_start(chunk, buf, sem, sidx):
    for k in range(ECH // 16):
      pv = packc_v[pl.ds(chunk * ECH + k * 16, 16)]
      sidx[pl.ds(k * 16, 16)] = pv & 16383
    pltpu.async_copy(x_hbm.at[sidx], buf, sem)

  def _g_wait(buf, sem, sidx):
    pltpu.make_async_copy(x_hbm.at[sidx], buf, sem).wait()

  def _scale(chunk, buf, sbuf):
    for k in range(ECH // 16):
      nv = ewnc_v[pl.ds(chunk * ECH + k * 16, 16)]
      for j in range(16):
        b = _bcast_lane(nv, j)
        row = k * 16 + j
        for r in range(IN_CH // 32):
          va = buf[row, pl.ds(r * 32, 16)] * b
          vb = buf[row, pl.ds(r * 32 + 16, 16)] * b
          sbuf[row, pl.ds(r * 32, 32)] = plsc.pack(
              va, vb, format=plsc.PackFormat.INTERLEAVED)

  def _s_start(chunk, sbuf, sem, dlrow):
    for k in range(ECH // 16):
      pv = packc_v[pl.ds(chunk * ECH + k * 16, 16)]
      dloc_v[dlrow, pl.ds(k * 16, 16)] = lax.shift_right_logical(pv, 14) - lo
    pltpu.async_copy(sbuf, agg_sh.at[dloc_v.at[dlrow]], sem, add=True)

  def _s_wait(sbuf, sem):
    pltpu.make_async_copy(sbuf, agg_sh.at[dloc_v.at[0]], sem).wait()

  @pl.when(nch64 > 0)
  def _():
    _g_start(0, rowsa_v, sema, sidxa_v)

  @pl.loop(0, lax.div(nch64 + 1, jnp.int32(2)))
  def _(i):
    a = 2 * i
    _g_wait(rowsa_v, sema, sidxa_v)

    @pl.when(a + 1 < nch64)
    def _():
      _g_start(a + 1, rowsb_v, semb, sidxb_v)

    @pl.when(i > 0)
    def _():
      _s_wait(scata_v, ssema)

    _scale(a, rowsa_v, scata_v)
    _s_start(a, scata_v, ssema, 0)

    @pl.when(a + 1 < nch64)
    def _():
      _g_wait(rowsb_v, semb, sidxb_v)

      @pl.when(a + 2 < nch64)
      def _():
        _g_start(a + 2, rowsa_v, sema, sidxa_v)

      @pl.when(i > 0)
      def _():
        _s_wait(scatb_v, ssemb)

      _scale(a + 1, rowsb_v, scatb_v)
      _s_start(a + 1, scatb_v, ssemb, 1)

  @pl.when(nch64 > 0)
  def _():
    _s_wait(scata_v, ssema)

  @pl.when(nch64 > 1)
  def _():
    _s_wait(scatb_v, ssemb)

  plsc.subcore_barrier()

  pltpu.sync_copy(agg_sh.at[pl.ds(s * CORE_SLICE, CORE_SLICE)],
                  out_hbm.at[pl.ds(c * HALF + s * CORE_SLICE, CORE_SLICE)])


def _sc_aggregate(src3, dst3, ew3, x_pad):
  mesh = plsc.VectorSubcoreMesh(core_axis_name="c", subcore_axis_name="s")
  return pl.kernel(
      _sc_body,
      out_type=(jax.ShapeDtypeStruct((N_PAD, IN_CH), jnp.bfloat16),
                jax.ShapeDtypeStruct((DEG_ROWS, 16), jnp.float32)),
      mesh=mesh,
      scratch_types=[
          pltpu.VMEM((SROWS, 16), jnp.int32),
          pltpu.VMEM((SROWS, 16), jnp.int32),
          pltpu.VMEM((SROWS, 16), jnp.float32),
          pltpu.VMEM((CAP,), jnp.int32),
          pltpu.VMEM((CAP,), jnp.float32),
          pltpu.VMEM((2, ECH), jnp.int32),
          pltpu.VMEM((ECH,), jnp.int32),
          pltpu.VMEM((ECH,), jnp.int32),
          pltpu.VMEM((DEG_ROWS, 16), jnp.float32),
          pltpu.VMEM((ECH, IN_CH), jnp.float32),
          pltpu.VMEM((ECH, IN_CH), jnp.float32),
          pltpu.VMEM((ECH, IN_CH), jnp.bfloat16),
          pltpu.VMEM((ECH, IN_CH), jnp.bfloat16),
          pltpu.VMEM((DEG_CHUNKS, 128), jnp.int32),
          pltpu.SemaphoreType.DMA,
          pltpu.SemaphoreType.DMA,
          pltpu.SemaphoreType.DMA,
          pltpu.SemaphoreType.DMA,
          pltpu.VMEM_SHARED((HALF, IN_CH), jnp.bfloat16),
          pltpu.VMEM_SHARED((DEG_ROWS, 16), jnp.float32),
      ],
      compiler_params=pltpu.CompilerParams(
          needs_layout_passes=False, use_tc_tiling_on_sc=False),
      name="tgcn_sc_aggregate",
  )(src3, dst3, ew3, x_pad)


def _dense_body(agg_ref, x_ref, dis_ref, wzp_ref, wzo_ref, whp_ref, who_ref,
                wlz_ref, wlh_ref,
                bz_ref, blz_ref, bh_ref, blh_ref, wo_ref, bo_ref, out_ref):
  agg = agg_ref[...].astype(jnp.float32)
  sx = (dis_ref[...] * dis_ref[...]) * x_ref[...]
  mzp = jnp.dot(wzp_ref[...], wlz_ref[...],
                preferred_element_type=jnp.float32)
  mzo = jnp.dot(wzo_ref[...], wlz_ref[...],
                preferred_element_type=jnp.float32)
  mhp = jnp.dot(whp_ref[...], wlh_ref[...],
                preferred_element_type=jnp.float32)
  mho = jnp.dot(who_ref[...], wlh_ref[...],
                preferred_element_type=jnp.float32)
  cz = jnp.dot(bz_ref[...], wlz_ref[...],
               preferred_element_type=jnp.float32) + blz_ref[...]
  ch = jnp.dot(bh_ref[...], wlh_ref[...],
               preferred_element_type=jnp.float32) + blh_ref[...]
  z = jax.nn.sigmoid(
      jnp.dot(agg, mzp, preferred_element_type=jnp.float32)
      + jnp.dot(sx, mzo, preferred_element_type=jnp.float32) + cz)
  ht = jnp.tanh(
      jnp.dot(agg, mhp, preferred_element_type=jnp.float32)
      + jnp.dot(sx, mho, preferred_element_type=jnp.float32) + ch)
  hn = jax.nn.relu((1.0 - z) * ht)
  out_ref[...] = (
      jnp.dot(hn, wo_ref[...], preferred_element_type=jnp.float32)
      + bo_ref[...])


def _dense(agg, x, dis, W_zp, W_zo, W_hp, W_ho, Wlz1, Wlh1,
           bz, blz, bh, blh, W_out, b_out):
  blk = 2000
  grid = (N_NODES // blk,)
  row_spec = pl.BlockSpec((blk, IN_CH), lambda i: (i, 0))
  full = lambda shape: pl.BlockSpec(shape, lambda i: (0,) * len(shape))
  return pl.pallas_call(
      _dense_body,
      grid=grid,
      in_specs=[
          row_spec, row_spec,
          pl.BlockSpec((blk, 1), lambda i: (i, 0)),
          full((IN_CH, IN_CH)), full((IN_CH, IN_CH)),
          full((IN_CH, IN_CH)), full((IN_CH, IN_CH)),
          full((IN_CH, IN_CH)), full((IN_CH, IN_CH)),
          full((1, IN_CH)), full((1, IN_CH)),
          full((1, IN_CH)), full((1, IN_CH)),
          full((IN_CH, OUT_SIZE)), full((1, OUT_SIZE)),
      ],
      out_specs=pl.BlockSpec((blk, OUT_SIZE), lambda i: (i, 0)),
      out_shape=jax.ShapeDtypeStruct((N_NODES, OUT_SIZE), jnp.float32),
  )(agg, x, dis, W_zp, W_zo, W_hp, W_ho, Wlz1, Wlh1,
    bz, blz, bh, blh, W_out, b_out)


@jax.jit
def kernel(x, edge_index, edge_weight, W_z, b_z, W_r, b_r, W_h, b_h,
           Wl_z, bl_z, Wl_r, bl_r, Wl_h, bl_h, W_out, b_out):
  del W_r, b_r, Wl_r, bl_r

  src3 = edge_index[0].astype(jnp.int32).reshape(N_TILES * SUBCH, SROWS, 16)
  dst3 = edge_index[1].astype(jnp.int32).reshape(N_TILES * SUBCH, SROWS, 16)
  ew3 = edge_weight.astype(jnp.float32).reshape(N_TILES * SUBCH, SROWS, 16)

  agg, dis = _sc_aggregate(src3, dst3, ew3, x)

  Wlz1 = Wl_z[:IN_CH]
  Wlh1 = Wl_h[:IN_CH]

  perm = np.empty((IN_CH,), np.int32)
  for r in range(IN_CH // 32):
    for l in range(16):
      perm[32 * r + 2 * l] = 32 * r + l
      perm[32 * r + 2 * l + 1] = 32 * r + 16 + l

  return _dense(agg[:N_NODES], x,
                dis.reshape(N_PAD, 1)[:N_NODES],
                W_z[perm], W_z, W_h[perm], W_h, Wlz1, Wlh1,
                b_z.reshape(1, IN_CH), bl_z.reshape(1, IN_CH),
                b_h.reshape(1, IN_CH), bl_h.reshape(1, IN_CH),
                W_out, b_out.reshape(1, OUT_SIZE))
